# Initial kernel scaffold; baseline (speedup 1.0000x reference)
import jax
import jax.numpy as jnp
from jax.experimental import pallas as pl


def kernel(x1, edge_index1, edge_type1, x2, edge_index2, edge_type2, batch_size, W_rel, W_root, b1, W_out, b_out, W_lin, b_lin):
    raise NotImplementedError("write your pallas kernel here")


# speedup vs baseline: 11.7872x; 11.7872x over previous
import functools

import jax
import jax.numpy as jnp
from jax import lax
from jax.experimental import pallas as pl
from jax.experimental.pallas import tpu as pltpu
from jax.experimental.pallas import tpu_sc as plsc

N = 10000
E = 320000
D = 128
H = 128
R = 9
OUT = 10
B = 50

NC = 2
NS = 16
NR = N * R
NRP = 90112
CPT = NRP // NS
EPT = E // NS
K = 80
NB = EPT // K
WCH = 80
NCH = N // WCH
BN = 2000
GRP = N // B


def _hrel_body(x_ref, w_ref, o_ref):
    o_ref[...] = jnp.dot(x_ref[...], w_ref[...],
                         preferred_element_type=jnp.float32)


def _hrel(xs, w_rel):
    return pl.pallas_call(
        _hrel_body,
        grid=(2, R, N // BN),
        in_specs=[
            pl.BlockSpec((None, BN, D), lambda g, r, i: (g, i, 0)),
            pl.BlockSpec((None, D, H), lambda g, r, i: (r, 0, 0)),
        ],
        out_specs=pl.BlockSpec((None, None, BN, H),
                               lambda g, r, i: (g, r, i, 0)),
        out_shape=jax.ShapeDtypeStruct((2, R, N, H), jnp.float32),
    )(xs, w_rel)


def _post_body(xs_ref, agg_ref, wroot_ref, b1_ref, wout_ref, wlin_ref,
               bout_ref, o_ref):
    ng = BN // GRP
    row = lax.broadcasted_iota(jnp.int32, (ng, BN), 0)
    col = lax.broadcasted_iota(jnp.int32, (ng, BN), 1) // GRP
    pmat = (row == col).astype(jnp.float32)
    zacc = None
    for g in range(2):
        h = jnp.maximum(
            agg_ref[g] + jnp.dot(xs_ref[g], wroot_ref[...],
                                 preferred_element_type=jnp.float32)
            + b1_ref[...], 0.0)
        out10 = jnp.dot(h, wout_ref[...],
                        preferred_element_type=jnp.float32) + bout_ref[...]
        pooled = jax.lax.dot_general(
            pmat, out10, (((1,), (0,)), ((), ())),
            precision=lax.Precision.HIGHEST,
            preferred_element_type=jnp.float32) * (1.0 / GRP)
        z = jnp.dot(pooled, wlin_ref[g * OUT:(g + 1) * OUT, :],
                    preferred_element_type=jnp.float32)
        zacc = z if zacc is None else zacc + z
    o_ref[0] = zacc


def _post(xs, agg, w_root, b1, w_out, w_lin, b_out):
    return pl.pallas_call(
        _post_body,
        grid=(N // BN,),
        in_specs=[
            pl.BlockSpec((2, BN, D), lambda i: (0, i, 0)),
            pl.BlockSpec((2, BN, H), lambda i: (0, i, 0)),
            pl.BlockSpec((D, H), lambda i: (0, 0)),
            pl.BlockSpec((1, H), lambda i: (0, 0)),
            pl.BlockSpec((H, OUT), lambda i: (0, 0)),
            pl.BlockSpec((2 * OUT, 1), lambda i: (0, 0)),
            pl.BlockSpec((1, OUT), lambda i: (0, 0)),
        ],
        out_specs=pl.BlockSpec((1, BN // GRP, 1), lambda i: (i, 0, 0)),
        out_shape=jax.ShapeDtypeStruct((N // BN, BN // GRP, 1), jnp.float32),
    )(xs, agg, w_root, b1, w_out, w_lin, b_out)


def _sc_body(src_hbm, dst_hbm, typ_hbm, hrel_hbm, agg_hbm,
             inv_sh, agg_sh,
             srcbuf, dstbuf, typbuf, invibuf,
             ones_v, inv_v, rows_v, zbuf, cbuf):
    g = lax.axis_index("c")
    w = lax.axis_index("s")
    goff = jnp.full((16,), g * (R * N), dtype=jnp.int32)
    ebase = g * E + w * EPT

    def z16(i, _):
        cbuf[pl.ds(i * 16, 16)] = jnp.zeros((16,), jnp.float32)
        return 0
    lax.fori_loop(0, CPT // 16, z16, 0)

    def zrow(i, _):
        zbuf[i // 8, pl.ds((i % 8) * 16, 16)] = jnp.zeros((16,), jnp.float32)
        return 0
    lax.fori_loop(0, WCH * H // 16, zrow, 0)

    def o16(i, _):
        ones_v[pl.ds(i * 16, 16)] = jnp.ones((16,), jnp.float32)
        return 0
    lax.fori_loop(0, K // 16, o16, 0)

    pltpu.sync_copy(cbuf, inv_sh.at[pl.ds(w * CPT, CPT)])

    def zagg(c, _):
        idx = c * NS + w

        @pl.when(idx < NCH)
        def _():
            pltpu.sync_copy(zbuf, agg_sh.at[pl.ds(idx * WCH, WCH)])
        return 0
    lax.fori_loop(0, (NCH + NS - 1) // NS, zagg, 0)

    plsc.subcore_barrier()

    def count_batch(b, _):
        base = ebase + b * K
        pltpu.sync_copy(dst_hbm.at[pl.ds(base, K)], dstbuf)
        pltpu.sync_copy(typ_hbm.at[pl.ds(base, K)], typbuf)

        def mkidx(i, _):
            sl = pl.ds(i * 16, 16)
            invibuf[sl] = dstbuf[sl] * R + typbuf[sl]
            return 0
        lax.fori_loop(0, K // 16, mkidx, 0)
        pltpu.sync_copy(ones_v, inv_sh.at[invibuf], add=True)
        return 0
    lax.fori_loop(0, NB, count_batch, 0)

    plsc.subcore_barrier()

    pltpu.sync_copy(inv_sh.at[pl.ds(w * CPT, CPT)], cbuf)

    def invert(i, _):
        sl = pl.ds(i * 16, 16)
        cbuf[sl] = 1.0 / jnp.maximum(cbuf[sl], 1.0)
        return 0
    lax.fori_loop(0, CPT // 16, invert, 0)
    pltpu.sync_copy(cbuf, inv_sh.at[pl.ds(w * CPT, CPT)])

    plsc.subcore_barrier()

    def agg_batch(b, _):
        base = ebase + b * K
        pltpu.sync_copy(src_hbm.at[pl.ds(base, K)], srcbuf)
        pltpu.sync_copy(dst_hbm.at[pl.ds(base, K)], dstbuf)
        pltpu.sync_copy(typ_hbm.at[pl.ds(base, K)], typbuf)

        def mkidx(i, _):
            sl = pl.ds(i * 16, 16)
            invibuf[sl] = dstbuf[sl] * R + typbuf[sl]
            srcbuf[sl] = typbuf[sl] * N + srcbuf[sl] + goff
            return 0
        lax.fori_loop(0, K // 16, mkidx, 0)

        pltpu.sync_copy(hrel_hbm.at[srcbuf], rows_v)
        pltpu.sync_copy(inv_sh.at[invibuf], inv_v)

        def scale(k, _):
            ib = plsc.load_gather(inv_v, [jnp.full((16,), k, jnp.int32)])
            for jj in range(H // 16):
                sl = pl.ds(jj * 16, 16)
                rows_v[k, sl] = rows_v[k, sl] * ib
            return 0
        lax.fori_loop(0, K, scale, 0)

        pltpu.sync_copy(rows_v, agg_sh.at[dstbuf], add=True)
        return 0
    lax.fori_loop(0, NB, agg_batch, 0)

    plsc.subcore_barrier()

    def wout(c, _):
        idx = c * NS + w

        @pl.when(idx < NCH)
        def _():
            sl = pl.ds(idx * WCH, WCH)
            pltpu.sync_copy(agg_sh.at[sl], zbuf)
            pltpu.sync_copy(zbuf, agg_hbm.at[g, sl])
        return 0
    lax.fori_loop(0, (NCH + NS - 1) // NS, wout, 0)


_sc_aggregate = functools.partial(
    pl.kernel,
    out_type=jax.ShapeDtypeStruct((2, N, H), jnp.float32),
    mesh=plsc.VectorSubcoreMesh(core_axis_name="c", subcore_axis_name="s",
                                num_cores=NC, num_subcores=NS),
    compiler_params=pltpu.CompilerParams(needs_layout_passes=False),
    scratch_types=[
        pltpu.VMEM_SHARED((NRP,), jnp.float32),
        pltpu.VMEM_SHARED((N, H), jnp.float32),
        pltpu.VMEM((K,), jnp.int32),
        pltpu.VMEM((K,), jnp.int32),
        pltpu.VMEM((K,), jnp.int32),
        pltpu.VMEM((K,), jnp.int32),
        pltpu.VMEM((K,), jnp.float32),
        pltpu.VMEM((K,), jnp.float32),
        pltpu.VMEM((K, H), jnp.float32),
        pltpu.VMEM((WCH, H), jnp.float32),
        pltpu.VMEM((CPT,), jnp.float32),
    ],
)(_sc_body)


def kernel(x1, edge_index1, edge_type1, x2, edge_index2, edge_type2,
           batch_size, W_rel, W_root, b1, W_out, b_out, W_lin, b_lin):
    xs = jnp.stack([x1, x2])
    src = jnp.concatenate([edge_index1[0], edge_index2[0]])
    dst = jnp.concatenate([edge_index1[1], edge_index2[1]])
    typ = jnp.concatenate([edge_type1, edge_type2])

    hrel = _hrel(xs, W_rel).reshape(2 * R * N, H)
    agg = _sc_aggregate(src, dst, typ, hrel)

    outp = _post(xs, agg, W_root, b1.reshape(1, H), W_out, W_lin,
                 b_out.reshape(1, OUT))
    z = outp.reshape(B)
    m = (jnp.asarray(batch_size) == B).astype(jnp.float32)
    return z * m + b_lin[0]


# speedup vs baseline: 27.4726x; 2.3307x over previous
import functools

import jax
import jax.numpy as jnp
from jax import lax
from jax.experimental import pallas as pl
from jax.experimental.pallas import tpu as pltpu
from jax.experimental.pallas import tpu_sc as plsc

N = 10000
E = 320000
D = 128
H = 128
R = 9
OUT = 10
B = 50

NC = 2
NS = 16
NR = N * R
NRP = 90112
CPT = NRP // NS
EPT = E // NS
K = 80
NB = EPT // K
CH = 10
WCH = 80
NCH = N // WCH
BN = 2000
GRP = N // B


def _hrel_body(x_ref, w_ref, o_ref):
    o_ref[...] = jnp.dot(x_ref[...], w_ref[...],
                         preferred_element_type=jnp.float32)


def _hrel(xs, w_rel):
    return pl.pallas_call(
        _hrel_body,
        grid=(2, R, N // BN),
        in_specs=[
            pl.BlockSpec((None, BN, D), lambda g, r, i: (g, i, 0)),
            pl.BlockSpec((None, D, H), lambda g, r, i: (r, 0, 0)),
        ],
        out_specs=pl.BlockSpec((None, None, BN, H),
                               lambda g, r, i: (g, r, i, 0)),
        out_shape=jax.ShapeDtypeStruct((2, R, N, H), jnp.float32),
    )(xs, w_rel)


def _post_body(xs_ref, agg_ref, wroot_ref, b1_ref, wout_ref, wlin_ref,
               bout_ref, o_ref):
    ng = BN // GRP
    row = lax.broadcasted_iota(jnp.int32, (ng, BN), 0)
    col = lax.broadcasted_iota(jnp.int32, (ng, BN), 1) // GRP
    pmat = (row == col).astype(jnp.float32)
    zacc = None
    for g in range(2):
        h = jnp.maximum(
            agg_ref[g] + jnp.dot(xs_ref[g], wroot_ref[...],
                                 preferred_element_type=jnp.float32)
            + b1_ref[...], 0.0)
        out10 = jnp.dot(h, wout_ref[...],
                        preferred_element_type=jnp.float32) + bout_ref[...]
        pooled = jax.lax.dot_general(
            pmat, out10, (((1,), (0,)), ((), ())),
            precision=lax.Precision.HIGHEST,
            preferred_element_type=jnp.float32) * (1.0 / GRP)
        z = jnp.dot(pooled, wlin_ref[g * OUT:(g + 1) * OUT, :],
                    preferred_element_type=jnp.float32)
        zacc = z if zacc is None else zacc + z
    o_ref[0] = zacc


def _post(xs, agg, w_root, b1, w_out, w_lin, b_out):
    return pl.pallas_call(
        _post_body,
        grid=(N // BN,),
        in_specs=[
            pl.BlockSpec((2, BN, D), lambda i: (0, i, 0)),
            pl.BlockSpec((2, BN, H), lambda i: (0, i, 0)),
            pl.BlockSpec((D, H), lambda i: (0, 0)),
            pl.BlockSpec((1, H), lambda i: (0, 0)),
            pl.BlockSpec((H, OUT), lambda i: (0, 0)),
            pl.BlockSpec((2 * OUT, 1), lambda i: (0, 0)),
            pl.BlockSpec((1, OUT), lambda i: (0, 0)),
        ],
        out_specs=pl.BlockSpec((1, BN // GRP, 1), lambda i: (i, 0, 0)),
        out_shape=jax.ShapeDtypeStruct((N // BN, BN // GRP, 1), jnp.float32),
    )(xs, agg, w_root, b1, w_out, w_lin, b_out)


def _sc_body(src_hbm, dst_hbm, typ_hbm, hrel_hbm, agg_hbm, idx_hbm,
             inv_sh, agg_sh,
             srcbuf, dstbuf, typbuf, idx3,
             ib0, ib1, dx0, dx1, invib,
             ones_v, inv0, inv1, rows0, rows1, cbuf,
             semx0, semx1, semr0, semr1, semi0, semi1, sems0, sems1, semc):
    g = lax.axis_index("c")
    w = lax.axis_index("s")
    goff = jnp.full((16,), g * (R * N), dtype=jnp.int32)
    ebase = g * E + w * EPT
    tbase = (g * NS + w) * NB

    def z16(i, _):
        cbuf[pl.ds(i * 16, 16)] = jnp.zeros((16,), jnp.float32)
        return 0
    lax.fori_loop(0, CPT // 16, z16, 0)

    def zrow(i, _):
        rows0[i // 8, pl.ds((i % 8) * 16, 16)] = jnp.zeros((16,), jnp.float32)
        return 0
    lax.fori_loop(0, K * H // 16, zrow, 0)

    def o16(i, _):
        ones_v[pl.ds(i * 16, 16)] = jnp.ones((16,), jnp.float32)
        return 0
    lax.fori_loop(0, K // 16, o16, 0)

    pltpu.sync_copy(cbuf, inv_sh.at[pl.ds(w * CPT, CPT)])

    def zagg(c, _):
        idx = c * NS + w

        @pl.when(idx < NCH)
        def _():
            pltpu.sync_copy(rows0, agg_sh.at[pl.ds(idx * WCH, WCH)])
        return 0
    lax.fori_loop(0, (NCH + NS - 1) // NS, zagg, 0)

    plsc.subcore_barrier()

    def count_chunk(c, _):
        base = ebase + c * (CH * K)
        pltpu.sync_copy(src_hbm.at[pl.ds(base, CH * K)], srcbuf)
        pltpu.sync_copy(dst_hbm.at[pl.ds(base, CH * K)], dstbuf)
        pltpu.sync_copy(typ_hbm.at[pl.ds(base, CH * K)], typbuf)

        def mkidx(i, _):
            j = i // (K // 16)
            o = (i % (K // 16)) * 16
            sl = pl.ds(i * 16, 16)
            dv = dstbuf[sl]
            tv = typbuf[sl]
            idx3[pl.ds(j * 3 * K + o, 16)] = tv * N + srcbuf[sl] + goff
            idx3[pl.ds(j * 3 * K + K + o, 16)] = dv * R + tv
            idx3[pl.ds(j * 3 * K + 2 * K + o, 16)] = dv
            return 0
        lax.fori_loop(0, CH * K // 16, mkidx, 0)

        def fire(j, _):
            def cp(i, _):
                invib[pl.ds(i * 16, 16)] = idx3[pl.ds(j * 3 * K + K + i * 16, 16)]
                return 0
            lax.fori_loop(0, K // 16, cp, 0)
            pltpu.sync_copy(ones_v, inv_sh.at[invib], add=True)
            return 0
        lax.fori_loop(0, CH, fire, 0)

        pltpu.sync_copy(idx3, idx_hbm.at[pl.ds((tbase + c * CH) * 3 * K,
                                               CH * 3 * K)])
        return 0
    lax.fori_loop(0, NB // CH, count_chunk, 0)

    plsc.subcore_barrier()

    pltpu.sync_copy(inv_sh.at[pl.ds(w * CPT, CPT)], cbuf)

    def invert(i, _):
        sl = pl.ds(i * 16, 16)
        cbuf[sl] = 1.0 / jnp.maximum(cbuf[sl], 1.0)
        return 0
    lax.fori_loop(0, CPT // 16, invert, 0)
    pltpu.sync_copy(cbuf, inv_sh.at[pl.ds(w * CPT, CPT)])

    plsc.subcore_barrier()

    def ixissue(b, ib, semx):
        pltpu.async_copy(idx_hbm.at[pl.ds((tbase + b) * 3 * K, 3 * K)], ib,
                         semx)

    def ixwait(b, ib, semx):
        pltpu.make_async_copy(idx_hbm.at[pl.ds((tbase + b) * 3 * K, 3 * K)],
                              ib, semx).wait()

    def gissue(ib, rows_b, inv_b, semr, semi):
        pltpu.async_copy(hrel_hbm.at[ib.at[pl.ds(0, K)]], rows_b, semr)
        pltpu.async_copy(inv_sh.at[ib.at[pl.ds(K, K)]], inv_b, semi)

    def gwait(ib, rows_b, inv_b, semr, semi):
        pltpu.make_async_copy(hrel_hbm.at[ib.at[pl.ds(0, K)]], rows_b,
                              semr).wait()
        pltpu.make_async_copy(inv_sh.at[ib.at[pl.ds(K, K)]], inv_b,
                              semi).wait()

    def dxcopy(ib, dx):
        def cp(i, _):
            dx[pl.ds(i * 16, 16)] = ib[pl.ds(2 * K + i * 16, 16)]
            return 0
        lax.fori_loop(0, K // 16, cp, 0)

    def sissue(rows_b, dx, sems):
        pltpu.async_copy(rows_b, agg_sh.at[dx], sems, add=True)

    def swait(rows_b, dx, sems):
        pltpu.make_async_copy(rows_b, agg_sh.at[dx], sems).wait()

    def scale(rows_b, inv_b):
        def body(k, _):
            ib = plsc.load_gather(inv_b, [jnp.full((16,), k, jnp.int32)])
            for jj in range(H // 16):
                sl = pl.ds(jj * 16, 16)
                rows_b[k, sl] = rows_b[k, sl] * ib
            return 0
        lax.fori_loop(0, K, body, 0)

    ixissue(0, ib0, semx0)
    ixissue(1, ib1, semx1)
    ixwait(0, ib0, semx0)
    gissue(ib0, rows0, inv0, semr0, semi0)

    def pipe(p, _):
        b0 = 2 * p
        b1 = 2 * p + 1
        gwait(ib0, rows0, inv0, semr0, semi0)
        dxcopy(ib0, dx0)

        @pl.when(b0 + 2 < NB)
        def _():
            ixissue(b0 + 2, ib0, semx0)
        ixwait(b1, ib1, semx1)

        @pl.when(p > 0)
        def _():
            swait(rows1, dx1, sems1)
        gissue(ib1, rows1, inv1, semr1, semi1)
        scale(rows0, inv0)
        sissue(rows0, dx0, sems0)

        gwait(ib1, rows1, inv1, semr1, semi1)
        dxcopy(ib1, dx1)

        @pl.when(b1 + 2 < NB)
        def _():
            ixissue(b1 + 2, ib1, semx1)

        @pl.when(b0 + 2 < NB)
        def _():
            ixwait(b0 + 2, ib0, semx0)
            swait(rows0, dx0, sems0)
            gissue(ib0, rows0, inv0, semr0, semi0)
        scale(rows1, inv1)
        sissue(rows1, dx1, sems1)
        return 0
    lax.fori_loop(0, NB // 2, pipe, 0)

    swait(rows0, dx0, sems0)
    swait(rows1, dx1, sems1)

    plsc.subcore_barrier()

    def wout(c, _):
        idx = c * NS + w

        @pl.when(idx < NCH)
        def _():
            sl = pl.ds(idx * WCH, WCH)
            pltpu.sync_copy(agg_sh.at[sl], rows0)
            pltpu.sync_copy(rows0, agg_hbm.at[g, sl])
        return 0
    lax.fori_loop(0, (NCH + NS - 1) // NS, wout, 0)


_sc_aggregate = functools.partial(
    pl.kernel,
    out_type=[jax.ShapeDtypeStruct((2, N, H), jnp.float32),
              jax.ShapeDtypeStruct((2 * NS * NB * 3 * K,), jnp.int32)],
    mesh=plsc.VectorSubcoreMesh(core_axis_name="c", subcore_axis_name="s",
                                num_cores=NC, num_subcores=NS),
    compiler_params=pltpu.CompilerParams(needs_layout_passes=False),
    scratch_types=[
        pltpu.VMEM_SHARED((NRP,), jnp.float32),
        pltpu.VMEM_SHARED((N, H), jnp.float32),
        pltpu.VMEM((CH * K,), jnp.int32),
        pltpu.VMEM((CH * K,), jnp.int32),
        pltpu.VMEM((CH * K,), jnp.int32),
        pltpu.VMEM((CH * 3 * K,), jnp.int32),
        pltpu.VMEM((3 * K,), jnp.int32),
        pltpu.VMEM((3 * K,), jnp.int32),
        pltpu.VMEM((K,), jnp.int32),
        pltpu.VMEM((K,), jnp.int32),
        pltpu.VMEM((K,), jnp.int32),
        pltpu.VMEM((K,), jnp.float32),
        pltpu.VMEM((K,), jnp.float32),
        pltpu.VMEM((K,), jnp.float32),
        pltpu.VMEM((K, H), jnp.float32),
        pltpu.VMEM((K, H), jnp.float32),
        pltpu.VMEM((CPT,), jnp.float32),
        pltpu.SemaphoreType.DMA,
        pltpu.SemaphoreType.DMA,
        pltpu.SemaphoreType.DMA,
        pltpu.SemaphoreType.DMA,
        pltpu.SemaphoreType.DMA,
        pltpu.SemaphoreType.DMA,
        pltpu.SemaphoreType.DMA,
        pltpu.SemaphoreType.DMA,
        pltpu.SemaphoreType.DMA,
    ],
)(_sc_body)


def kernel(x1, edge_index1, edge_type1, x2, edge_index2, edge_type2,
           batch_size, W_rel, W_root, b1, W_out, b_out, W_lin, b_lin):
    xs = jnp.stack([x1, x2])
    src = jnp.concatenate([edge_index1[0], edge_index2[0]])
    dst = jnp.concatenate([edge_index1[1], edge_index2[1]])
    typ = jnp.concatenate([edge_type1, edge_type2])

    hrel = _hrel(xs, W_rel).reshape(2 * R * N, H)
    agg, _ = _sc_aggregate(src, dst, typ, hrel)

    outp = _post(xs, agg, W_root, b1.reshape(1, H), W_out, W_lin,
                 b_out.reshape(1, OUT))
    z = outp.reshape(B)
    m = (jnp.asarray(batch_size) == B).astype(jnp.float32)
    return z * m + b_lin[0]


# speedup vs baseline: 30.1111x; 1.0960x over previous
import functools

import jax
import jax.numpy as jnp
from jax import lax
from jax.experimental import pallas as pl
from jax.experimental.pallas import tpu as pltpu
from jax.experimental.pallas import tpu_sc as plsc

N = 10000
E = 320000
D = 128
H = 128
R = 9
OUT = 10
B = 50

NC = 2
NS = 16
NR = N * R
NRP = 90112
CPT = NRP // NS
EPT = E // NS
K = 80
NB = EPT // K
CH = 10
WCH = 80
NCH = N // WCH
BN = 2000
GRP = N // B


def _hrel_body(x_ref, w_ref, o_ref):
    o_ref[...] = jnp.dot(x_ref[...], w_ref[...],
                         preferred_element_type=jnp.float32)


def _hrel(xs, w_rel):
    return pl.pallas_call(
        _hrel_body,
        grid=(2, R, N // BN),
        in_specs=[
            pl.BlockSpec((None, BN, D), lambda g, r, i: (g, i, 0)),
            pl.BlockSpec((None, D, H), lambda g, r, i: (r, 0, 0)),
        ],
        out_specs=pl.BlockSpec((None, None, BN, H),
                               lambda g, r, i: (g, r, i, 0)),
        out_shape=jax.ShapeDtypeStruct((2, R, N, H), jnp.float32),
    )(xs, w_rel)


def _post_body(xs_ref, agg_ref, wroot_ref, b1_ref, wout_ref, wlin_ref,
               bout_ref, o_ref):
    ng = BN // GRP
    row = lax.broadcasted_iota(jnp.int32, (ng, BN), 0)
    col = lax.broadcasted_iota(jnp.int32, (ng, BN), 1) // GRP
    pmat = (row == col).astype(jnp.float32)
    zacc = None
    for g in range(2):
        h = jnp.maximum(
            agg_ref[g] + jnp.dot(xs_ref[g], wroot_ref[...],
                                 preferred_element_type=jnp.float32)
            + b1_ref[...], 0.0)
        out10 = jnp.dot(h, wout_ref[...],
                        preferred_element_type=jnp.float32) + bout_ref[...]
        pooled = jax.lax.dot_general(
            pmat, out10, (((1,), (0,)), ((), ())),
            precision=lax.Precision.HIGHEST,
            preferred_element_type=jnp.float32) * (1.0 / GRP)
        z = jnp.dot(pooled, wlin_ref[g * OUT:(g + 1) * OUT, :],
                    preferred_element_type=jnp.float32)
        zacc = z if zacc is None else zacc + z
    o_ref[0] = zacc


def _post(xs, agg, w_root, b1, w_out, w_lin, b_out):
    return pl.pallas_call(
        _post_body,
        grid=(N // BN,),
        in_specs=[
            pl.BlockSpec((2, BN, D), lambda i: (0, i, 0)),
            pl.BlockSpec((2, BN, H), lambda i: (0, i, 0)),
            pl.BlockSpec((D, H), lambda i: (0, 0)),
            pl.BlockSpec((1, H), lambda i: (0, 0)),
            pl.BlockSpec((H, OUT), lambda i: (0, 0)),
            pl.BlockSpec((2 * OUT, 1), lambda i: (0, 0)),
            pl.BlockSpec((1, OUT), lambda i: (0, 0)),
        ],
        out_specs=pl.BlockSpec((1, BN // GRP, 1), lambda i: (i, 0, 0)),
        out_shape=jax.ShapeDtypeStruct((N // BN, BN // GRP, 1), jnp.float32),
    )(xs, agg, w_root, b1, w_out, w_lin, b_out)


def _sc_body(src_hbm, dst_hbm, typ_hbm, hrel_hbm, agg_hbm, idx_hbm,
             inv_sh, agg_sh,
             srcbuf, dstbuf, typbuf, idx3,
             ib0, ib1, dx0, dx1, invib0, invib1,
             ones_v, inv0, inv1, rows0, rows1, cbuf,
             semx0, semx1, semr0, semr1, semi0, semi1, sems0, sems1,
             semc0, semc1):
    g = lax.axis_index("c")
    w = lax.axis_index("s")
    goff = jnp.full((16,), g * (R * N), dtype=jnp.int32)
    ebase = g * E + w * EPT
    tbase = (g * NS + w) * NB

    def z16(i, _):
        cbuf[pl.ds(i * 16, 16)] = jnp.zeros((16,), jnp.float32)
        return 0
    lax.fori_loop(0, CPT // 16, z16, 0)

    def zrow(i, _):
        rows0[i // 8, pl.ds((i % 8) * 16, 16)] = jnp.zeros((16,), jnp.float32)
        return 0
    lax.fori_loop(0, K * H // 16, zrow, 0)

    def o16(i, _):
        ones_v[pl.ds(i * 16, 16)] = jnp.ones((16,), jnp.float32)
        return 0
    lax.fori_loop(0, K // 16, o16, 0)

    pltpu.sync_copy(cbuf, inv_sh.at[pl.ds(w * CPT, CPT)])

    def zagg(c, _):
        idx = c * NS + w

        @pl.when(idx < NCH)
        def _():
            pltpu.sync_copy(rows0, agg_sh.at[pl.ds(idx * WCH, WCH)])
        return 0
    lax.fori_loop(0, (NCH + NS - 1) // NS, zagg, 0)

    plsc.subcore_barrier()

    def count_chunk(c, _):
        base = ebase + c * (CH * K)
        pltpu.sync_copy(src_hbm.at[pl.ds(base, CH * K)], srcbuf)
        pltpu.sync_copy(dst_hbm.at[pl.ds(base, CH * K)], dstbuf)
        pltpu.sync_copy(typ_hbm.at[pl.ds(base, CH * K)], typbuf)

        @plsc.parallel_loop(0, CH * K // 16, unroll=2)
        def mkidx(i):
            j = i // (K // 16)
            o = (i % (K // 16)) * 16
            sl = pl.ds(i * 16, 16)
            dv = dstbuf[sl]
            tv = typbuf[sl]
            idx3[pl.ds(j * 3 * K + o, 16)] = tv * N + srcbuf[sl] + goff
            idx3[pl.ds(j * 3 * K + K + o, 16)] = dv * R + tv
            idx3[pl.ds(j * 3 * K + 2 * K + o, 16)] = dv

        for j in range(CH):
            bb = invib0 if j % 2 == 0 else invib1
            sem = semc0 if j % 2 == 0 else semc1
            if j < 2:
                @pl.when(c > 0)
                def _(bb=bb, sem=sem):
                    pltpu.make_async_copy(ones_v, inv_sh.at[bb], sem).wait()
            else:
                pltpu.make_async_copy(ones_v, inv_sh.at[bb], sem).wait()

            @plsc.parallel_loop(0, K // 16)
            def cp(i, j=j, bb=bb):
                bb[pl.ds(i * 16, 16)] = idx3[pl.ds(j * 3 * K + K + i * 16,
                                                   16)]
            pltpu.async_copy(ones_v, inv_sh.at[bb], sem, add=True)

        pltpu.sync_copy(idx3, idx_hbm.at[pl.ds((tbase + c * CH) * 3 * K,
                                               CH * 3 * K)])
        return 0
    lax.fori_loop(0, NB // CH, count_chunk, 0)
    pltpu.make_async_copy(ones_v, inv_sh.at[invib0], semc0).wait()
    pltpu.make_async_copy(ones_v, inv_sh.at[invib1], semc1).wait()

    plsc.subcore_barrier()

    pltpu.sync_copy(inv_sh.at[pl.ds(w * CPT, CPT)], cbuf)

    def invert(i, _):
        sl = pl.ds(i * 16, 16)
        cbuf[sl] = 1.0 / jnp.maximum(cbuf[sl], 1.0)
        return 0
    lax.fori_loop(0, CPT // 16, invert, 0)
    pltpu.sync_copy(cbuf, inv_sh.at[pl.ds(w * CPT, CPT)])

    plsc.subcore_barrier()

    def ixissue(b, ib, semx):
        pltpu.async_copy(idx_hbm.at[pl.ds((tbase + b) * 3 * K, 3 * K)], ib,
                         semx)

    def ixwait(b, ib, semx):
        pltpu.make_async_copy(idx_hbm.at[pl.ds((tbase + b) * 3 * K, 3 * K)],
                              ib, semx).wait()

    def gissue(ib, rows_b, inv_b, semr, semi):
        pltpu.async_copy(hrel_hbm.at[ib.at[pl.ds(0, K)]], rows_b, semr)
        pltpu.async_copy(inv_sh.at[ib.at[pl.ds(K, K)]], inv_b, semi)

    def gwait(ib, rows_b, inv_b, semr, semi):
        pltpu.make_async_copy(hrel_hbm.at[ib.at[pl.ds(0, K)]], rows_b,
                              semr).wait()
        pltpu.make_async_copy(inv_sh.at[ib.at[pl.ds(K, K)]], inv_b,
                              semi).wait()

    def dxcopy(ib, dx):
        def cp(i, _):
            dx[pl.ds(i * 16, 16)] = ib[pl.ds(2 * K + i * 16, 16)]
            return 0
        lax.fori_loop(0, K // 16, cp, 0)

    def sissue(rows_b, dx, sems):
        pltpu.async_copy(rows_b, agg_sh.at[dx], sems, add=True)

    def swait(rows_b, dx, sems):
        pltpu.make_async_copy(rows_b, agg_sh.at[dx], sems).wait()

    def scale(rows_b, inv_b):
        @plsc.parallel_loop(0, K, unroll=4)
        def body(k):
            ib = plsc.load_gather(inv_b, [jnp.full((16,), k, jnp.int32)])
            for jj in range(H // 16):
                sl = pl.ds(jj * 16, 16)
                rows_b[k, sl] = rows_b[k, sl] * ib

    ixissue(0, ib0, semx0)
    ixissue(1, ib1, semx1)
    ixwait(0, ib0, semx0)
    gissue(ib0, rows0, inv0, semr0, semi0)

    def pipe(p, _):
        b0 = 2 * p
        b1 = 2 * p + 1
        gwait(ib0, rows0, inv0, semr0, semi0)
        dxcopy(ib0, dx0)

        @pl.when(b0 + 2 < NB)
        def _():
            ixissue(b0 + 2, ib0, semx0)
        ixwait(b1, ib1, semx1)

        @pl.when(p > 0)
        def _():
            swait(rows1, dx1, sems1)
        gissue(ib1, rows1, inv1, semr1, semi1)
        scale(rows0, inv0)
        sissue(rows0, dx0, sems0)

        gwait(ib1, rows1, inv1, semr1, semi1)
        dxcopy(ib1, dx1)

        @pl.when(b1 + 2 < NB)
        def _():
            ixissue(b1 + 2, ib1, semx1)

        @pl.when(b0 + 2 < NB)
        def _():
            ixwait(b0 + 2, ib0, semx0)
            swait(rows0, dx0, sems0)
            gissue(ib0, rows0, inv0, semr0, semi0)
        scale(rows1, inv1)
        sissue(rows1, dx1, sems1)
        return 0
    lax.fori_loop(0, NB // 2, pipe, 0)

    swait(rows0, dx0, sems0)
    swait(rows1, dx1, sems1)

    plsc.subcore_barrier()

    def wout(c, _):
        idx = c * NS + w

        @pl.when(idx < NCH)
        def _():
            sl = pl.ds(idx * WCH, WCH)
            pltpu.sync_copy(agg_sh.at[sl], rows0)
            pltpu.sync_copy(rows0, agg_hbm.at[g, sl])
        return 0
    lax.fori_loop(0, (NCH + NS - 1) // NS, wout, 0)


_sc_aggregate = functools.partial(
    pl.kernel,
    out_type=[jax.ShapeDtypeStruct((2, N, H), jnp.float32),
              jax.ShapeDtypeStruct((2 * NS * NB * 3 * K,), jnp.int32)],
    mesh=plsc.VectorSubcoreMesh(core_axis_name="c", subcore_axis_name="s",
                                num_cores=NC, num_subcores=NS),
    compiler_params=pltpu.CompilerParams(needs_layout_passes=False),
    scratch_types=[
        pltpu.VMEM_SHARED((NRP,), jnp.float32),
        pltpu.VMEM_SHARED((N, H), jnp.float32),
        pltpu.VMEM((CH * K,), jnp.int32),
        pltpu.VMEM((CH * K,), jnp.int32),
        pltpu.VMEM((CH * K,), jnp.int32),
        pltpu.VMEM((CH * 3 * K,), jnp.int32),
        pltpu.VMEM((3 * K,), jnp.int32),
        pltpu.VMEM((3 * K,), jnp.int32),
        pltpu.VMEM((K,), jnp.int32),
        pltpu.VMEM((K,), jnp.int32),
        pltpu.VMEM((K,), jnp.int32),
        pltpu.VMEM((K,), jnp.int32),
        pltpu.VMEM((K,), jnp.float32),
        pltpu.VMEM((K,), jnp.float32),
        pltpu.VMEM((K,), jnp.float32),
        pltpu.VMEM((K, H), jnp.float32),
        pltpu.VMEM((K, H), jnp.float32),
        pltpu.VMEM((CPT,), jnp.float32),
        pltpu.SemaphoreType.DMA,
        pltpu.SemaphoreType.DMA,
        pltpu.SemaphoreType.DMA,
        pltpu.SemaphoreType.DMA,
        pltpu.SemaphoreType.DMA,
        pltpu.SemaphoreType.DMA,
        pltpu.SemaphoreType.DMA,
        pltpu.SemaphoreType.DMA,
        pltpu.SemaphoreType.DMA,
        pltpu.SemaphoreType.DMA,
    ],
)(_sc_body)


def kernel(x1, edge_index1, edge_type1, x2, edge_index2, edge_type2,
           batch_size, W_rel, W_root, b1, W_out, b_out, W_lin, b_lin):
    xs = jnp.stack([x1, x2])
    src = jnp.concatenate([edge_index1[0], edge_index2[0]])
    dst = jnp.concatenate([edge_index1[1], edge_index2[1]])
    typ = jnp.concatenate([edge_type1, edge_type2])

    hrel = _hrel(xs, W_rel).reshape(2 * R * N, H)
    agg, _ = _sc_aggregate(src, dst, typ, hrel)

    outp = _post(xs, agg, W_root, b1.reshape(1, H), W_out, W_lin,
                 b_out.reshape(1, OUT))
    z = outp.reshape(B)
    m = (jnp.asarray(batch_size) == B).astype(jnp.float32)
    return z * m + b_lin[0]
